# baseline (speedup 1.0000x reference)
import functools

import jax
import jax.numpy as jnp
from jax import lax
from jax.experimental import pallas as pl
from jax.experimental.pallas import tpu as pltpu
from jax.experimental.pallas import tpu_sc as plsc

C = 128
NUM_CORES = 2
NUM_SUBCORES = 16
NW = NUM_CORES * NUM_SUBCORES
UNROLL = 4


def _transform_kernel(x_ref, w_ref, o_ref):
    o_ref[...] = lax.dot_general(
        x_ref[...], w_ref[0],
        dimension_numbers=(((1,), (1,)), ((), ())),
        preferred_element_type=jnp.float32,
    )


def _gidx_kernel(n_nodes, src_ref, et_ref, o_ref):
    o_ref[...] = et_ref[...] * n_nodes + src_ref[...]


def _combine_kernel(p0_ref, p1_ref, b_ref, o_ref):
    o_ref[...] = p0_ref[0] + p1_ref[0] + b_ref[...]


def _sc_body(t0_per_sub, t1_per_sub, rows_per_sub,
             h_ref, gd_ref, out_ref,
             idx0, idx1, idx2, idx3, rows0, rows1, acc,
             isem0, isem1, isem2, isem3, gsem0, gsem1, ssem0, ssem1):
    c = lax.axis_index("c")
    s = lax.axis_index("s")
    row0 = s * rows_per_sub
    t_per_sub = jnp.where(c == 0, t0_per_sub, t1_per_sub)
    base = c * NUM_SUBCORES * t0_per_sub + s * t_per_sub

    idxs = [idx0, idx1, idx2, idx3]
    isems = [isem0, isem1, isem2, isem3]
    rows = [rows0, rows1]
    gsems = [gsem0, gsem1]
    ssems = [ssem0, ssem1]

    @pl.loop(0, C)
    def _(i):
        for m in range(8):
            rows0[i, pl.ds(m * 16, 16)] = jnp.zeros((16,), jnp.float32)

    zoffs = list(range(0, rows_per_sub, C))
    zcps = [pltpu.async_copy(
                rows0.at[pl.ds(0, min(C, rows_per_sub - o))],
                acc.at[pl.ds(row0 + o, min(C, rows_per_sub - o))],
                isems[i % 4])
            for i, o in enumerate(zoffs)]
    for cp in zcps:
        cp.wait()
    plsc.subcore_barrier()

    def start_load(t, q):
        pltpu.async_copy(gd_ref.at[base + t], idxs[q], isems[q])

    def wait_load(q):
        pltpu.make_async_copy(gd_ref.at[base], idxs[q], isems[q]).wait()

    def start_gather(q, b):
        pltpu.async_copy(h_ref.at[idxs[q].at[0]], rows[b], gsems[b])

    def wait_gather(q, b):
        pltpu.make_async_copy(h_ref.at[idxs[q].at[0]], rows[b],
                              gsems[b]).wait()

    def start_scatter(q, b):
        pltpu.async_copy(rows[b], acc.at[idxs[q].at[1]], ssems[b], add=True)

    def wait_scatter(q, b):
        pltpu.make_async_copy(rows[b], acc.at[idxs[q].at[1]],
                              ssems[b]).wait()

    def steady(t, q):
        b = q % 2
        wait_load(q)
        wait_scatter((q + 2) % 4, b)
        start_gather(q, b)
        start_load(t + 2, (q + 2) % 4)
        wait_gather((q + 3) % 4, b ^ 1)
        start_scatter((q + 3) % 4, b ^ 1)

    _sc_pipeline(t_per_sub, start_load, wait_load, start_gather,
                 wait_gather, start_scatter, wait_scatter, steady)

    plsc.subcore_barrier()
    wb = -(-rows_per_sub // 64) * 8
    offs = list(range(0, rows_per_sub, wb))
    cps = [pltpu.async_copy(
               acc.at[pl.ds(row0 + o, min(wb, rows_per_sub - o))],
               out_ref.at[c, pl.ds(row0 + o, min(wb, rows_per_sub - o))],
               isems[i % 4])
           for i, o in enumerate(offs)]
    for cp in cps:
        cp.wait()


def _sc_pipeline(t_per_sub, start_load, wait_load, start_gather, wait_gather,
                 start_scatter, wait_scatter, steady):
    start_load(0, 0)
    start_load(1, 1)
    wait_load(0)
    start_gather(0, 0)
    start_load(2, 2)
    wait_load(1)
    start_gather(1, 1)
    start_load(3, 3)
    wait_gather(0, 0)
    start_scatter(0, 0)
    wait_load(2)
    wait_scatter(0, 0)
    start_gather(2, 0)
    start_load(4, 0)
    wait_gather(1, 1)
    start_scatter(1, 1)

    wait_load(3)
    wait_scatter(1, 1)
    start_gather(3, 1)
    start_load(5, 1)
    wait_gather(2, 0)
    start_scatter(2, 0)

    @pl.loop(1, t_per_sub // UNROLL)
    def _(j):
        t0 = j * UNROLL
        steady(t0 + 0, 0)
        steady(t0 + 1, 1)
        steady(t0 + 2, 2)
        steady(t0 + 3, 3)

    last_q = 3
    last_b = 1
    wait_gather(last_q, last_b)
    start_scatter(last_q, last_b)
    wait_scatter((last_q + 3) % 4, last_b ^ 1)
    wait_scatter(last_q, last_b)
    wait_load(0)
    wait_load(1)


def kernel(node_features, node_repr, edge_index, edge_types, num_relations,
           weight, bias):
    del node_features, num_relations
    n = node_repr.shape[0]
    d_in = node_repr.shape[1]
    r = weight.shape[0]
    d_out = weight.shape[1]
    e = edge_types.shape[0]

    bn = 1000
    assert n % bn == 0
    h_all = pl.pallas_call(
        _transform_kernel,
        grid=(r, n // bn),
        in_specs=[
            pl.BlockSpec((bn, d_in), lambda ri, ni: (ni, 0)),
            pl.BlockSpec((1, d_out, d_in), lambda ri, ni: (ri, 0, 0)),
        ],
        out_specs=pl.BlockSpec((bn, d_out), lambda ri, ni: (ri * (n // bn) + ni, 0)),
        out_shape=jax.ShapeDtypeStruct((r * n, d_out), jnp.float32),
    )(node_repr, weight)

    chunks_total = -(-e // (C * NW * UNROLL)) * NW * UNROLL
    e_pad = chunks_total * C
    pad = e_pad - e
    src_p = jnp.concatenate(
        [edge_index[0], jnp.zeros((pad,), jnp.int32)]).reshape(chunks_total, C)
    et_p = jnp.concatenate(
        [edge_types, jnp.zeros((pad,), jnp.int32)]).reshape(chunks_total, C)
    dst_p = jnp.concatenate(
        [edge_index[1], jnp.full((pad,), n, jnp.int32)]).reshape(chunks_total, C)

    bc = chunks_total // 10
    g_p = pl.pallas_call(
        functools.partial(_gidx_kernel, n),
        grid=(10,),
        in_specs=[
            pl.BlockSpec((bc, C), lambda i: (i, 0)),
            pl.BlockSpec((bc, C), lambda i: (i, 0)),
        ],
        out_specs=pl.BlockSpec((bc, C), lambda i: (i, 0)),
        out_shape=jax.ShapeDtypeStruct((chunks_total, C), jnp.int32),
    )(src_p, et_p)

    gd = jnp.concatenate(
        [jnp.stack([g_p, dst_p], axis=1),
         jnp.zeros((2, 2, C), jnp.int32)], axis=0)

    t_all = chunks_total // NUM_SUBCORES
    t0_per_sub = t_all - 16
    t1_per_sub = t_all - t0_per_sub
    assert t0_per_sub % UNROLL == 0 and t1_per_sub % UNROLL == 0
    assert t0_per_sub >= 2 * UNROLL and t1_per_sub >= 2 * UNROLL
    rows_per_sub = -(-(n + 1) // (NUM_SUBCORES * 8)) * 8
    n_pad = rows_per_sub * NUM_SUBCORES

    mesh = plsc.VectorSubcoreMesh(core_axis_name="c", subcore_axis_name="s")
    sc_kernel = pl.kernel(
        functools.partial(_sc_body, t0_per_sub, t1_per_sub, rows_per_sub),
        out_type=jax.ShapeDtypeStruct((NUM_CORES, n_pad, d_out), jnp.float32),
        mesh=mesh,
        scratch_types=[
            pltpu.VMEM((2, C), jnp.int32),
            pltpu.VMEM((2, C), jnp.int32),
            pltpu.VMEM((2, C), jnp.int32),
            pltpu.VMEM((2, C), jnp.int32),
            pltpu.VMEM((C, 128), jnp.float32),
            pltpu.VMEM((C, 128), jnp.float32),
            pltpu.VMEM_SHARED((n_pad, 128), jnp.float32),
            pltpu.SemaphoreType.DMA,
            pltpu.SemaphoreType.DMA,
            pltpu.SemaphoreType.DMA,
            pltpu.SemaphoreType.DMA,
            pltpu.SemaphoreType.DMA,
            pltpu.SemaphoreType.DMA,
            pltpu.SemaphoreType.DMA,
            pltpu.SemaphoreType.DMA,
        ],
    )
    partials = sc_kernel(h_all, gd)

    out = pl.pallas_call(
        _combine_kernel,
        grid=(n // bn,),
        in_specs=[
            pl.BlockSpec((1, bn, d_out), lambda i: (0, i, 0)),
            pl.BlockSpec((1, bn, d_out), lambda i: (1, i, 0)),
            pl.BlockSpec((1, d_out), lambda i: (0, 0)),
        ],
        out_specs=pl.BlockSpec((bn, d_out), lambda i: (i, 0)),
        out_shape=jax.ShapeDtypeStruct((n, d_out), jnp.float32),
    )(partials, partials, bias.reshape(1, d_out))

    return out


# speedup vs baseline: 1.0113x; 1.0113x over previous
import functools

import jax
import jax.numpy as jnp
from jax import lax
from jax.experimental import pallas as pl
from jax.experimental.pallas import tpu as pltpu
from jax.experimental.pallas import tpu_sc as plsc

C = 128
NUM_CORES = 2
NUM_SUBCORES = 16
NW = NUM_CORES * NUM_SUBCORES
UNROLL = 4


def _transform_kernel(x_ref, w_ref, o_ref):
    o_ref[...] = lax.dot_general(
        x_ref[...], w_ref[0],
        dimension_numbers=(((1,), (1,)), ((), ())),
        preferred_element_type=jnp.float32,
    )


def _gidx_kernel(n_nodes, src_ref, et_ref, o_ref):
    o_ref[...] = et_ref[...] * n_nodes + src_ref[...]


def _combine_kernel(p0_ref, p1_ref, b_ref, o_ref):
    o_ref[...] = p0_ref[0] + p1_ref[0] + b_ref[...]


def _sc_body(t0_per_sub, t1_per_sub, rows_per_sub,
             h_ref, gd_ref, out_ref,
             idx0, idx1, idx2, idx3, rows0, rows1, acc,
             isem0, isem1, isem2, isem3, gsem0, gsem1, ssem0, ssem1):
    c = lax.axis_index("c")
    s = lax.axis_index("s")
    row0 = s * rows_per_sub
    t_per_sub = jnp.where(c == 0, t0_per_sub, t1_per_sub)
    base = c * NUM_SUBCORES * t0_per_sub + s * t_per_sub

    idxs = [idx0, idx1, idx2, idx3]
    isems = [isem0, isem1, isem2, isem3]
    rows = [rows0, rows1]
    gsems = [gsem0, gsem1]
    ssems = [ssem0, ssem1]

    @pl.loop(0, C)
    def _(i):
        for m in range(8):
            rows0[i, pl.ds(m * 16, 16)] = jnp.zeros((16,), jnp.float32)

    zoffs = list(range(0, rows_per_sub, C))
    zcps = [pltpu.async_copy(
                rows0.at[pl.ds(0, min(C, rows_per_sub - o))],
                acc.at[pl.ds(row0 + o, min(C, rows_per_sub - o))],
                isems[i % 4])
            for i, o in enumerate(zoffs)]
    for cp in zcps:
        cp.wait()
    plsc.subcore_barrier()

    def start_load(t, q):
        pltpu.async_copy(gd_ref.at[base + t], idxs[q], isems[q])

    def wait_load(q):
        pltpu.make_async_copy(gd_ref.at[base], idxs[q], isems[q]).wait()

    def start_gather(q, b):
        pltpu.async_copy(h_ref.at[idxs[q].at[0]], rows[b], gsems[b])

    def wait_gather(q, b):
        pltpu.make_async_copy(h_ref.at[idxs[q].at[0]], rows[b],
                              gsems[b]).wait()

    def start_scatter(q, b):
        pltpu.async_copy(rows[b], acc.at[idxs[q].at[1]], ssems[b], add=True)

    def wait_scatter(q, b):
        pltpu.make_async_copy(rows[b], acc.at[idxs[q].at[1]],
                              ssems[b]).wait()

    def steady(t, q):
        b = q % 2
        wait_load(q)
        wait_scatter((q + 2) % 4, b)
        start_gather(q, b)
        start_load(t + 2, (q + 2) % 4)
        wait_gather((q + 3) % 4, b ^ 1)
        start_scatter((q + 3) % 4, b ^ 1)

    _sc_pipeline(t_per_sub, start_load, wait_load, start_gather,
                 wait_gather, start_scatter, wait_scatter, steady)

    plsc.subcore_barrier()
    wb = -(-rows_per_sub // 64) * 8
    offs = list(range(0, rows_per_sub, wb))
    cps = [pltpu.async_copy(
               acc.at[pl.ds(row0 + o, min(wb, rows_per_sub - o))],
               out_ref.at[c, pl.ds(row0 + o, min(wb, rows_per_sub - o))],
               isems[i % 4])
           for i, o in enumerate(offs)]
    for cp in cps:
        cp.wait()


def _sc_pipeline(t_per_sub, start_load, wait_load, start_gather, wait_gather,
                 start_scatter, wait_scatter, steady):
    start_load(0, 0)
    start_load(1, 1)
    wait_load(0)
    start_gather(0, 0)
    start_load(2, 2)
    wait_load(1)
    start_gather(1, 1)
    start_load(3, 3)
    wait_gather(0, 0)
    start_scatter(0, 0)
    wait_load(2)
    wait_scatter(0, 0)
    start_gather(2, 0)
    start_load(4, 0)
    wait_gather(1, 1)
    start_scatter(1, 1)

    wait_load(3)
    wait_scatter(1, 1)
    start_gather(3, 1)
    start_load(5, 1)
    wait_gather(2, 0)
    start_scatter(2, 0)

    @pl.loop(1, t_per_sub // UNROLL)
    def _(j):
        t0 = j * UNROLL
        steady(t0 + 0, 0)
        steady(t0 + 1, 1)
        steady(t0 + 2, 2)
        steady(t0 + 3, 3)

    last_q = 3
    last_b = 1
    wait_gather(last_q, last_b)
    start_scatter(last_q, last_b)
    wait_scatter((last_q + 3) % 4, last_b ^ 1)
    wait_scatter(last_q, last_b)
    wait_load(0)
    wait_load(1)


def kernel(node_features, node_repr, edge_index, edge_types, num_relations,
           weight, bias):
    del node_features, num_relations
    n = node_repr.shape[0]
    d_in = node_repr.shape[1]
    r = weight.shape[0]
    d_out = weight.shape[1]
    e = edge_types.shape[0]

    bn = 1000
    assert n % bn == 0
    h_all = pl.pallas_call(
        _transform_kernel,
        grid=(r, n // bn),
        in_specs=[
            pl.BlockSpec((bn, d_in), lambda ri, ni: (ni, 0)),
            pl.BlockSpec((1, d_out, d_in), lambda ri, ni: (ri, 0, 0)),
        ],
        out_specs=pl.BlockSpec((bn, d_out), lambda ri, ni: (ri * (n // bn) + ni, 0)),
        out_shape=jax.ShapeDtypeStruct((r * n, d_out), jnp.float32),
    )(node_repr, weight)

    chunks_total = -(-e // (C * NW * UNROLL)) * NW * UNROLL
    e_pad = chunks_total * C
    pad = e_pad - e
    src_p = jnp.concatenate(
        [edge_index[0], jnp.zeros((pad,), jnp.int32)]).reshape(chunks_total, C)
    et_p = jnp.concatenate(
        [edge_types, jnp.zeros((pad,), jnp.int32)]).reshape(chunks_total, C)
    dst_p = jnp.concatenate(
        [edge_index[1], jnp.full((pad,), n, jnp.int32)]).reshape(chunks_total, C)

    bc = chunks_total // 10
    g_p = pl.pallas_call(
        functools.partial(_gidx_kernel, n),
        grid=(10,),
        in_specs=[
            pl.BlockSpec((bc, C), lambda i: (i, 0)),
            pl.BlockSpec((bc, C), lambda i: (i, 0)),
        ],
        out_specs=pl.BlockSpec((bc, C), lambda i: (i, 0)),
        out_shape=jax.ShapeDtypeStruct((chunks_total, C), jnp.int32),
    )(src_p, et_p)

    gd = jnp.concatenate(
        [jnp.stack([g_p, dst_p], axis=1),
         jnp.zeros((2, 2, C), jnp.int32)], axis=0)

    t_all = chunks_total // NUM_SUBCORES
    t0_per_sub = t_all - 8
    t1_per_sub = t_all - t0_per_sub
    assert t0_per_sub % UNROLL == 0 and t1_per_sub % UNROLL == 0
    assert t0_per_sub >= 2 * UNROLL and t1_per_sub >= 2 * UNROLL
    rows_per_sub = -(-(n + 1) // (NUM_SUBCORES * 8)) * 8
    n_pad = rows_per_sub * NUM_SUBCORES

    mesh = plsc.VectorSubcoreMesh(core_axis_name="c", subcore_axis_name="s")
    sc_kernel = pl.kernel(
        functools.partial(_sc_body, t0_per_sub, t1_per_sub, rows_per_sub),
        out_type=jax.ShapeDtypeStruct((NUM_CORES, n_pad, d_out), jnp.float32),
        mesh=mesh,
        scratch_types=[
            pltpu.VMEM((2, C), jnp.int32),
            pltpu.VMEM((2, C), jnp.int32),
            pltpu.VMEM((2, C), jnp.int32),
            pltpu.VMEM((2, C), jnp.int32),
            pltpu.VMEM((C, 128), jnp.float32),
            pltpu.VMEM((C, 128), jnp.float32),
            pltpu.VMEM_SHARED((n_pad, 128), jnp.float32),
            pltpu.SemaphoreType.DMA,
            pltpu.SemaphoreType.DMA,
            pltpu.SemaphoreType.DMA,
            pltpu.SemaphoreType.DMA,
            pltpu.SemaphoreType.DMA,
            pltpu.SemaphoreType.DMA,
            pltpu.SemaphoreType.DMA,
            pltpu.SemaphoreType.DMA,
        ],
    )
    partials = sc_kernel(h_all, gd)

    out = pl.pallas_call(
        _combine_kernel,
        grid=(n // bn,),
        in_specs=[
            pl.BlockSpec((1, bn, d_out), lambda i: (0, i, 0)),
            pl.BlockSpec((1, bn, d_out), lambda i: (1, i, 0)),
            pl.BlockSpec((1, d_out), lambda i: (0, 0)),
        ],
        out_specs=pl.BlockSpec((bn, d_out), lambda i: (i, 0)),
        out_shape=jax.ShapeDtypeStruct((n, d_out), jnp.float32),
    )(partials, partials, bias.reshape(1, d_out))

    return out
